# Initial kernel scaffold; baseline (speedup 1.0000x reference)
import jax
import jax.numpy as jnp
from jax.experimental import pallas as pl


def kernel(pos, atomic_numbers, batch, molecule_size, emb, W1, b1, W2, b2, Wlin):
    raise NotImplementedError("write your pallas kernel here")


# speedup vs baseline: 27.4844x; 27.4844x over previous
import math

import jax
import jax.numpy as jnp
import numpy as np
from jax.experimental import pallas as pl
from jax.experimental.pallas import tpu as pltpu

_M = 128
_S = 8
_N = _M * _S
_HS = 128
_K = 32
_L = 5
_SH = 25
_CUT = 15.0
_ALPHA = 0.5
_NUM_TYPES = 20

_NB = 128
_MB = _NB // _S
_G = _N // _NB

_LOGBINOM = np.log(
    np.array([math.comb(_K - 1, k) for k in range(_K)], dtype=np.float64)
).astype(np.float32)


def _edge_geometry(pos, logb):
    p3 = pos.reshape(_MB, _S, 3)
    kk = jax.lax.broadcasted_iota(jnp.int32, (1, _K), 1).astype(jnp.float32)
    rbf_l, shT_l, valid_l = [], [], []
    for d in range(1, _S):
        ps = jnp.roll(p3, -d, axis=1).reshape(_NB, 3)
        ev = ps - pos
        ex = ev[:, 0:1]
        ey = ev[:, 1:2]
        ez = ev[:, 2:3]
        r2 = ex * ex + ey * ey + ez * ez
        r = jnp.sqrt(r2)
        valid = (r < _CUT).astype(jnp.float32)
        xb = jnp.exp(-_ALPHA * r)
        logx = jnp.log(xb + 1e-10)
        log1mx = jnp.log(1.0 - xb + 1e-10)
        fcut = jnp.where(
            r < _CUT, jnp.exp(-r2 / ((_CUT - r) * (_CUT + r) + 1e-9)), 0.0
        )
        rbf = jnp.exp(logb + kk * logx + (_K - 1 - kk) * log1mx) * fcut
        rinv = 1.0 / (r + 1e-9)
        x = ey * rinv
        y = ez * rinv
        z = ex * rinv
        x2 = x * x
        y2 = y * y
        z2 = z * z
        s3 = math.sqrt(3.0)
        s5 = math.sqrt(5.0)
        s15 = math.sqrt(15.0)
        a = math.sqrt(35.0 / 8.0)
        b = math.sqrt(105.0)
        c = math.sqrt(21.0 / 8.0)
        dd = math.sqrt(7.0) / 2.0
        e = 0.75 * math.sqrt(35.0)
        f = 0.75 * math.sqrt(17.5)
        g = 0.75 * s5
        h = 0.75 * math.sqrt(2.5)
        cols = [
            jnp.ones_like(x),
            s3 * x,
            s3 * y,
            s3 * z,
            s15 * x * y,
            s15 * y * z,
            0.5 * s5 * (3 * z2 - 1),
            s15 * x * z,
            0.5 * s15 * (x2 - y2),
            a * y * (3 * x2 - y2),
            b * x * y * z,
            c * y * (5 * z2 - 1),
            dd * (5 * z2 - 3) * z,
            c * x * (5 * z2 - 1),
            0.5 * b * z * (x2 - y2),
            a * x * (x2 - y2),
            e * x * y * (x2 - y2),
            f * y * z * (3 * x2 - y2),
            g * x * y * (7 * z2 - 1),
            h * y * z * (7 * z2 - 3),
            0.375 * (35 * z2 * z2 - 30 * z2 + 3),
            h * x * z * (7 * z2 - 3),
            0.375 * s5 * (x2 - y2) * (7 * z2 - 1),
            f * x * z * (x2 - y2),
            (3.0 / 16.0) * math.sqrt(35.0) * (x2 * x2 - 6 * x2 * y2 + y2 * y2),
        ]
        sh = jnp.concatenate(cols, axis=1)
        rbf_l.append(rbf)
        shT_l.append(jnp.transpose(sh))
        valid_l.append(valid)
    return rbf_l, shT_l, valid_l


def _body(pos_ref, an_ref, logb_ref, emb_ref, W1_ref, b1_ref, W2_ref, b2_ref,
          Wlin_ref, out_ref):
    pos = pos_ref[:]
    rbf_l, shT_l, valid_l = _edge_geometry(pos, logb_ref[:])
    rbf_all = jnp.concatenate(rbf_l, axis=0)
    valid_all = jnp.concatenate(valid_l, axis=0)

    an = an_ref[:]
    tt = jax.lax.broadcasted_iota(jnp.int32, (1, _NUM_TYPES), 1)
    oh = (an == tt).astype(jnp.float32)
    node_attr = jnp.dot(oh, emb_ref[:], preferred_element_type=jnp.float32)

    X0 = jnp.concatenate(
        [node_attr[None], jnp.zeros((_SH - 1, _NB, _HS), jnp.float32)], axis=0
    )

    inv = 1.0 / float(_S - 1)

    def layer(i, X):
        h = jnp.maximum(
            jnp.dot(rbf_all, W1_ref[i], preferred_element_type=jnp.float32)
            + b1_ref[i],
            0.0,
        )
        w = (
            jnp.dot(h, W2_ref[i], preferred_element_type=jnp.float32)
            + b2_ref[i]
        ) * valid_all
        acc = None
        for di in range(_S - 1):
            d = di + 1
            w1g = w[di * _NB:(di + 1) * _NB, :_HS]
            w2g = w[di * _NB:(di + 1) * _NB, _HS:]
            Xr = jnp.roll(
                X.reshape(_SH, _MB, _S, _HS), -d, axis=2
            ).reshape(_SH, _NB, _HS)
            t = w1g[None] * Xr + shT_l[di][:, :, None] * w2g[None]
            acc = t if acc is None else acc + t
        agg = acc * inv
        new = jnp.dot(
            agg.reshape(_SH * _NB, _HS),
            Wlin_ref[i],
            preferred_element_type=jnp.float32,
        ).reshape(_SH, _NB, _HS)
        alpha = jnp.where(i > 0, 1.0, 0.0).astype(jnp.float32)
        xn = alpha * X + new
        sc = xn[0]
        gate = jax.nn.sigmoid(sc)
        head = jax.nn.softplus(sc) - math.log(2.0)
        return jnp.concatenate([head[None], xn[1:] * gate[None]], axis=0)

    out_ref[:] = jax.lax.fori_loop(0, _L, layer, X0)


def kernel(pos, atomic_numbers, batch, molecule_size, emb, W1, b1, W2, b2,
           Wlin):
    del batch, molecule_size
    an2 = atomic_numbers.reshape(_N, 1).astype(jnp.int32)
    b1r = b1.reshape(_L, 1, _HS)
    b2r = b2.reshape(_L, 1, 2 * _HS)
    logb = jnp.asarray(_LOGBINOM).reshape(1, _K)
    res = pl.pallas_call(
        _body,
        grid=(_G,),
        in_specs=[
            pl.BlockSpec((_NB, 3), lambda g: (g, 0)),
            pl.BlockSpec((_NB, 1), lambda g: (g, 0)),
            pl.BlockSpec((1, _K), lambda g: (0, 0)),
            pl.BlockSpec((_NUM_TYPES, _HS), lambda g: (0, 0)),
            pl.BlockSpec((_L, _K, _HS), lambda g: (0, 0, 0)),
            pl.BlockSpec((_L, 1, _HS), lambda g: (0, 0, 0)),
            pl.BlockSpec((_L, _HS, 2 * _HS), lambda g: (0, 0, 0)),
            pl.BlockSpec((_L, 1, 2 * _HS), lambda g: (0, 0, 0)),
            pl.BlockSpec((_L, _HS, _HS), lambda g: (0, 0, 0)),
        ],
        out_specs=pl.BlockSpec((_SH, _NB, _HS), lambda g: (0, g, 0)),
        out_shape=jax.ShapeDtypeStruct((_SH, _N, _HS), jnp.float32),
    )(pos, an2, logb, emb, W1, b1r, W2, b2r, Wlin)
    return jnp.transpose(res, (1, 2, 0))


# speedup vs baseline: 30.8458x; 1.1223x over previous
import math

import jax
import jax.numpy as jnp
import numpy as np
from jax.experimental import pallas as pl

_M = 128
_S = 8
_N = _M * _S
_HS = 128
_K = 32
_L = 5
_SH = 25
_CUT = 15.0
_ALPHA = 0.5
_NUM_TYPES = 20

_MB = 16
_NB = _MB * _S
_G = _N // _NB
_E = (_S - 1) * _NB

_LOGBINOM = np.log(
    np.array([math.comb(_K - 1, k) for k in range(_K)], dtype=np.float64)
).astype(np.float32)


def _col(rows_7x128):
    parts = [jnp.transpose(rows_7x128[d:d + 1, :]) for d in range(_S - 1)]
    return jnp.concatenate(parts, axis=0)


def _edge_geometry(posT, logb):
    ev_rows = []
    for d in range(1, _S):
        src = jnp.roll(posT, -_MB * d, axis=1)
        ev_rows.append(src - posT)
    ex = jnp.concatenate([e[0:1] for e in ev_rows], axis=0)
    ey = jnp.concatenate([e[1:2] for e in ev_rows], axis=0)
    ez = jnp.concatenate([e[2:3] for e in ev_rows], axis=0)
    r2 = ex * ex + ey * ey + ez * ez
    r = jnp.sqrt(r2)
    valid = (r < _CUT).astype(jnp.float32)
    xb = jnp.exp(-_ALPHA * r)
    logx = jnp.log(xb + 1e-10)
    log1mx = jnp.log(1.0 - xb + 1e-10)
    fcut = jnp.where(
        r < _CUT, jnp.exp(-r2 / ((_CUT - r) * (_CUT + r) + 1e-9)), 0.0
    )
    rinv = 1.0 / (r + 1e-9)
    x = ey * rinv
    y = ez * rinv
    z = ex * rinv
    x2 = x * x
    y2 = y * y
    z2 = z * z
    s3 = math.sqrt(3.0)
    s5 = math.sqrt(5.0)
    s15 = math.sqrt(15.0)
    a_ = math.sqrt(35.0 / 8.0)
    b_ = math.sqrt(105.0)
    c_ = math.sqrt(21.0 / 8.0)
    dd = math.sqrt(7.0) / 2.0
    e_ = 0.75 * math.sqrt(35.0)
    f_ = 0.75 * math.sqrt(17.5)
    g_ = 0.75 * s5
    h_ = 0.75 * math.sqrt(2.5)
    comps = [
        jnp.ones_like(x),
        s3 * x,
        s3 * y,
        s3 * z,
        s15 * x * y,
        s15 * y * z,
        0.5 * s5 * (3 * z2 - 1),
        s15 * x * z,
        0.5 * s15 * (x2 - y2),
        a_ * y * (3 * x2 - y2),
        b_ * x * y * z,
        c_ * y * (5 * z2 - 1),
        dd * (5 * z2 - 3) * z,
        c_ * x * (5 * z2 - 1),
        0.5 * b_ * z * (x2 - y2),
        a_ * x * (x2 - y2),
        e_ * x * y * (x2 - y2),
        f_ * y * z * (3 * x2 - y2),
        g_ * x * y * (7 * z2 - 1),
        h_ * y * z * (7 * z2 - 3),
        0.375 * (35 * z2 * z2 - 30 * z2 + 3),
        h_ * x * z * (7 * z2 - 3),
        0.375 * s5 * (x2 - y2) * (7 * z2 - 1),
        f_ * x * z * (x2 - y2),
        (3.0 / 16.0) * math.sqrt(35.0) * (x2 * x2 - 6 * x2 * y2 + y2 * y2),
    ]
    shW = jnp.stack(comps, axis=0)
    sh_sl = [
        [shW[:, d, a * _MB:(a + 1) * _MB] for a in range(_S)]
        for d in range(_S - 1)
    ]

    kk = jax.lax.broadcasted_iota(jnp.int32, (1, _K), 1).astype(jnp.float32)
    logxC = _col(logx)
    log1mxC = _col(log1mx)
    fcutC = _col(fcut)
    validC = _col(valid)
    rbf = jnp.exp(logb + kk * logxC + (_K - 1 - kk) * log1mxC) * fcutC
    return rbf, validC, sh_sl


def _body(posT_ref, an_ref, logb_ref, emb_ref, W1_ref, b1_ref, W2_ref,
          b2_ref, Wlin_ref, out_ref):
    rbf, validC, sh_sl = _edge_geometry(posT_ref[:], logb_ref[:])

    an = an_ref[:]
    tt = jax.lax.broadcasted_iota(jnp.int32, (1, _NUM_TYPES), 1)
    oh = (an == tt).astype(jnp.float32)
    node_attr = jnp.dot(oh, emb_ref[:], preferred_element_type=jnp.float32)

    zeros_tail = jnp.zeros((_SH - 1, _MB, _HS), jnp.float32)
    X0 = tuple(
        jnp.concatenate(
            [node_attr[None, a * _MB:(a + 1) * _MB, :], zeros_tail], axis=0
        )
        for a in range(_S)
    )

    inv = 1.0 / float(_S - 1)

    def layer(i, X):
        h = jnp.maximum(
            jnp.dot(rbf, W1_ref[i], preferred_element_type=jnp.float32)
            + b1_ref[i],
            0.0,
        )
        w = (
            jnp.dot(h, W2_ref[i], preferred_element_type=jnp.float32)
            + b2_ref[i]
        ) * validC
        Xn = []
        for a in range(_S):
            acc = None
            for d in range(1, _S):
                b = (a + d) % _S
                row = (d - 1) * _NB + a * _MB
                w1g = w[row:row + _MB, :_HS]
                w2g = w[row:row + _MB, _HS:]
                t = w1g[None] * X[b] + sh_sl[d - 1][a][:, :, None] * w2g[None]
                acc = t if acc is None else acc + t
            new = jnp.dot(
                (acc * inv).reshape(_SH * _MB, _HS),
                Wlin_ref[i],
                preferred_element_type=jnp.float32,
            ).reshape(_SH, _MB, _HS)
            Xn.append(new)
        alpha = jnp.where(i > 0, 1.0, 0.0).astype(jnp.float32)
        out = []
        for a in range(_S):
            xn = alpha * X[a] + Xn[a]
            sc = xn[0]
            gate = jax.nn.sigmoid(sc)
            head = jax.nn.softplus(sc) - math.log(2.0)
            out.append(
                jnp.concatenate([head[None], xn[1:] * gate[None]], axis=0)
            )
        return tuple(out)

    Xf = jax.lax.fori_loop(0, _L, layer, X0)
    for a in range(_S):
        out_ref[:, a * _MB:(a + 1) * _MB, :] = Xf[a]


def kernel(pos, atomic_numbers, batch, molecule_size, emb, W1, b1, W2, b2,
           Wlin):
    del batch, molecule_size
    posT = jnp.transpose(
        pos.reshape(_G, _MB, _S, 3), (3, 0, 2, 1)
    ).reshape(3, _N)
    anP = jnp.transpose(
        atomic_numbers.reshape(_G, _MB, _S), (0, 2, 1)
    ).reshape(_N, 1).astype(jnp.int32)
    b1r = b1.reshape(_L, 1, _HS)
    b2r = b2.reshape(_L, 1, 2 * _HS)
    logb = jnp.asarray(_LOGBINOM).reshape(1, _K)
    res = pl.pallas_call(
        _body,
        grid=(_G,),
        in_specs=[
            pl.BlockSpec((3, _NB), lambda g: (0, g)),
            pl.BlockSpec((_NB, 1), lambda g: (g, 0)),
            pl.BlockSpec((1, _K), lambda g: (0, 0)),
            pl.BlockSpec((_NUM_TYPES, _HS), lambda g: (0, 0)),
            pl.BlockSpec((_L, _K, _HS), lambda g: (0, 0, 0)),
            pl.BlockSpec((_L, 1, _HS), lambda g: (0, 0, 0)),
            pl.BlockSpec((_L, _HS, 2 * _HS), lambda g: (0, 0, 0)),
            pl.BlockSpec((_L, 1, 2 * _HS), lambda g: (0, 0, 0)),
            pl.BlockSpec((_L, _HS, _HS), lambda g: (0, 0, 0)),
        ],
        out_specs=pl.BlockSpec((_SH, _NB, _HS), lambda g: (0, g, 0)),
        out_shape=jax.ShapeDtypeStruct((_SH, _N, _HS), jnp.float32),
    )(posT, anP, logb, emb, W1, b1r, W2, b2r, Wlin)
    return jnp.transpose(
        res.reshape(_SH, _G, _S, _MB, _HS), (1, 3, 2, 4, 0)
    ).reshape(_N, _HS, _SH)


# speedup vs baseline: 36.1461x; 1.1718x over previous
import math

import jax
import jax.numpy as jnp
import numpy as np
from jax.experimental import pallas as pl
from jax.experimental.pallas import tpu as pltpu

_M = 128
_S = 8
_N = _M * _S
_HS = 128
_K = 32
_L = 5
_SH = 25
_CUT = 15.0
_ALPHA = 0.5
_NUM_TYPES = 20

_MB = 16
_NB = _MB * _S
_G = _N // _NB
_E = (_S - 1) * _NB

_LOGBINOM = np.log(
    np.array([math.comb(_K - 1, k) for k in range(_K)], dtype=np.float64)
).astype(np.float32)


def _col(rows_7x128):
    parts = [jnp.transpose(rows_7x128[d:d + 1, :]) for d in range(_S - 1)]
    return jnp.concatenate(parts, axis=0)


def _edge_geometry(posT, logb):
    ev_rows = []
    for d in range(1, _S):
        src = jnp.roll(posT, -_MB * d, axis=1)
        ev_rows.append(src - posT)
    ex = jnp.concatenate([e[0:1] for e in ev_rows], axis=0)
    ey = jnp.concatenate([e[1:2] for e in ev_rows], axis=0)
    ez = jnp.concatenate([e[2:3] for e in ev_rows], axis=0)
    r2 = ex * ex + ey * ey + ez * ez
    r = jnp.sqrt(r2)
    valid = (r < _CUT).astype(jnp.float32)
    xb = jnp.exp(-_ALPHA * r)
    logx = jnp.log(xb + 1e-10)
    log1mx = jnp.log(1.0 - xb + 1e-10)
    fcut = jnp.where(
        r < _CUT, jnp.exp(-r2 / ((_CUT - r) * (_CUT + r) + 1e-9)), 0.0
    )
    rinv = 1.0 / (r + 1e-9)
    x = ey * rinv
    y = ez * rinv
    z = ex * rinv
    x2 = x * x
    y2 = y * y
    z2 = z * z
    s3 = math.sqrt(3.0)
    s5 = math.sqrt(5.0)
    s15 = math.sqrt(15.0)
    a_ = math.sqrt(35.0 / 8.0)
    b_ = math.sqrt(105.0)
    c_ = math.sqrt(21.0 / 8.0)
    dd = math.sqrt(7.0) / 2.0
    e_ = 0.75 * math.sqrt(35.0)
    f_ = 0.75 * math.sqrt(17.5)
    g_ = 0.75 * s5
    h_ = 0.75 * math.sqrt(2.5)
    comps = [
        jnp.ones_like(x),
        s3 * x,
        s3 * y,
        s3 * z,
        s15 * x * y,
        s15 * y * z,
        0.5 * s5 * (3 * z2 - 1),
        s15 * x * z,
        0.5 * s15 * (x2 - y2),
        a_ * y * (3 * x2 - y2),
        b_ * x * y * z,
        c_ * y * (5 * z2 - 1),
        dd * (5 * z2 - 3) * z,
        c_ * x * (5 * z2 - 1),
        0.5 * b_ * z * (x2 - y2),
        a_ * x * (x2 - y2),
        e_ * x * y * (x2 - y2),
        f_ * y * z * (3 * x2 - y2),
        g_ * x * y * (7 * z2 - 1),
        h_ * y * z * (7 * z2 - 3),
        0.375 * (35 * z2 * z2 - 30 * z2 + 3),
        h_ * x * z * (7 * z2 - 3),
        0.375 * s5 * (x2 - y2) * (7 * z2 - 1),
        f_ * x * z * (x2 - y2),
        (3.0 / 16.0) * math.sqrt(35.0) * (x2 * x2 - 6 * x2 * y2 + y2 * y2),
    ]
    shW = jnp.stack(comps, axis=0)
    sh_sl = [
        [
            jnp.broadcast_to(
                shW[:, d, a * _MB:(a + 1) * _MB][:, :, None],
                (_SH, _MB, _HS),
            )
            for a in range(_S)
        ]
        for d in range(_S - 1)
    ]

    kk = jax.lax.broadcasted_iota(jnp.int32, (1, _K), 1).astype(jnp.float32)
    logxC = _col(logx)
    log1mxC = _col(log1mx)
    fcutC = _col(fcut)
    validC = _col(valid)
    rbf = jnp.exp(logb + kk * logxC + (_K - 1 - kk) * log1mxC) * fcutC
    return rbf, validC, sh_sl


def _body(posT_ref, an_ref, logb_ref, emb_ref, W1_ref, b1_ref, W2_ref,
          b2_ref, Wlin_ref, out_ref):
    rbf, validC, sh_sl = _edge_geometry(posT_ref[:], logb_ref[:])

    an = an_ref[:]
    tt = jax.lax.broadcasted_iota(jnp.int32, (1, _NUM_TYPES), 1)
    oh = (an == tt).astype(jnp.float32)
    node_attr = jnp.dot(oh, emb_ref[:], preferred_element_type=jnp.float32)

    zeros_tail = jnp.zeros((_SH - 1, _MB, _HS), jnp.float32)
    X0 = tuple(
        jnp.concatenate(
            [node_attr[None, a * _MB:(a + 1) * _MB, :], zeros_tail], axis=0
        )
        for a in range(_S)
    )

    inv = 1.0 / float(_S - 1)

    def layer(i, X):
        h = jnp.maximum(
            jnp.dot(rbf, W1_ref[i], preferred_element_type=jnp.float32)
            + b1_ref[i],
            0.0,
        )
        w = (
            jnp.dot(h, W2_ref[i], preferred_element_type=jnp.float32)
            + b2_ref[i]
        ) * validC
        Xn = []
        for a in range(_S):
            acc = None
            for d in range(1, _S):
                b = (a + d) % _S
                row = (d - 1) * _NB + a * _MB
                w1g = w[row:row + _MB, :_HS]
                w2g = w[row:row + _MB, _HS:]
                t = w1g[None] * X[b] + sh_sl[d - 1][a] * w2g[None]
                acc = t if acc is None else acc + t
            new = jnp.dot(
                (acc * inv).reshape(_SH * _MB, _HS),
                Wlin_ref[i],
                preferred_element_type=jnp.float32,
            ).reshape(_SH, _MB, _HS)
            Xn.append(new)
        alpha = jnp.where(i > 0, 1.0, 0.0).astype(jnp.float32)
        out = []
        for a in range(_S):
            xn = alpha * X[a] + Xn[a]
            sc = xn[0]
            gate = jax.nn.sigmoid(sc)
            head = jax.nn.softplus(sc) - math.log(2.0)
            out.append(
                jnp.concatenate([head[None], xn[1:] * gate[None]], axis=0)
            )
        return tuple(out)

    Xf = jax.lax.fori_loop(0, _L, layer, X0)
    for a in range(_S):
        out_ref[:, a * _MB:(a + 1) * _MB, :] = Xf[a]


def kernel(pos, atomic_numbers, batch, molecule_size, emb, W1, b1, W2, b2,
           Wlin):
    del batch, molecule_size
    posT = jnp.transpose(
        pos.reshape(_G, _MB, _S, 3), (3, 0, 2, 1)
    ).reshape(3, _N)
    anP = jnp.transpose(
        atomic_numbers.reshape(_G, _MB, _S), (0, 2, 1)
    ).reshape(_N, 1).astype(jnp.int32)
    b1r = b1.reshape(_L, 1, _HS)
    b2r = b2.reshape(_L, 1, 2 * _HS)
    logb = jnp.asarray(_LOGBINOM).reshape(1, _K)
    res = pl.pallas_call(
        _body,
        grid=(_G,),
        in_specs=[
            pl.BlockSpec((3, _NB), lambda g: (0, g)),
            pl.BlockSpec((_NB, 1), lambda g: (g, 0)),
            pl.BlockSpec((1, _K), lambda g: (0, 0)),
            pl.BlockSpec((_NUM_TYPES, _HS), lambda g: (0, 0)),
            pl.BlockSpec((_L, _K, _HS), lambda g: (0, 0, 0)),
            pl.BlockSpec((_L, 1, _HS), lambda g: (0, 0, 0)),
            pl.BlockSpec((_L, _HS, 2 * _HS), lambda g: (0, 0, 0)),
            pl.BlockSpec((_L, 1, 2 * _HS), lambda g: (0, 0, 0)),
            pl.BlockSpec((_L, _HS, _HS), lambda g: (0, 0, 0)),
        ],
        out_specs=pl.BlockSpec((_SH, _NB, _HS), lambda g: (0, g, 0)),
        out_shape=jax.ShapeDtypeStruct((_SH, _N, _HS), jnp.float32),
        compiler_params=pltpu.CompilerParams(
            dimension_semantics=("parallel",)
        ),
    )(posT, anP, logb, emb, W1, b1r, W2, b2r, Wlin)
    return jnp.transpose(
        res.reshape(_SH, _G, _S, _MB, _HS), (1, 3, 2, 4, 0)
    ).reshape(_N, _HS, _SH)


# speedup vs baseline: 37.1177x; 1.0269x over previous
import math

import jax
import jax.numpy as jnp
import numpy as np
from jax.experimental import pallas as pl
from jax.experimental.pallas import tpu as pltpu

_M = 128
_S = 8
_N = _M * _S
_HS = 128
_K = 32
_L = 5
_SH = 25
_CUT = 15.0
_ALPHA = 0.5
_NUM_TYPES = 20

_MB = 32
_NB = _MB * _S
_G = _N // _NB
_E = (_S - 1) * _NB

_LOGBINOM = np.log(
    np.array([math.comb(_K - 1, k) for k in range(_K)], dtype=np.float64)
).astype(np.float32)


def _col(rows_7x128):
    parts = [jnp.transpose(rows_7x128[d:d + 1, :]) for d in range(_S - 1)]
    return jnp.concatenate(parts, axis=0)


def _edge_geometry(posT, logb):
    ev_rows = []
    for d in range(1, _S):
        src = jnp.roll(posT, -_MB * d, axis=1)
        ev_rows.append(src - posT)
    ex = jnp.concatenate([e[0:1] for e in ev_rows], axis=0)
    ey = jnp.concatenate([e[1:2] for e in ev_rows], axis=0)
    ez = jnp.concatenate([e[2:3] for e in ev_rows], axis=0)
    r2 = ex * ex + ey * ey + ez * ez
    r = jnp.sqrt(r2)
    valid = (r < _CUT).astype(jnp.float32)
    xb = jnp.exp(-_ALPHA * r)
    logx = jnp.log(xb + 1e-10)
    log1mx = jnp.log(1.0 - xb + 1e-10)
    fcut = jnp.where(
        r < _CUT, jnp.exp(-r2 / ((_CUT - r) * (_CUT + r) + 1e-9)), 0.0
    )
    rinv = 1.0 / (r + 1e-9)
    x = ey * rinv
    y = ez * rinv
    z = ex * rinv
    x2 = x * x
    y2 = y * y
    z2 = z * z
    s3 = math.sqrt(3.0)
    s5 = math.sqrt(5.0)
    s15 = math.sqrt(15.0)
    a_ = math.sqrt(35.0 / 8.0)
    b_ = math.sqrt(105.0)
    c_ = math.sqrt(21.0 / 8.0)
    dd = math.sqrt(7.0) / 2.0
    e_ = 0.75 * math.sqrt(35.0)
    f_ = 0.75 * math.sqrt(17.5)
    g_ = 0.75 * s5
    h_ = 0.75 * math.sqrt(2.5)
    comps = [
        jnp.ones_like(x),
        s3 * x,
        s3 * y,
        s3 * z,
        s15 * x * y,
        s15 * y * z,
        0.5 * s5 * (3 * z2 - 1),
        s15 * x * z,
        0.5 * s15 * (x2 - y2),
        a_ * y * (3 * x2 - y2),
        b_ * x * y * z,
        c_ * y * (5 * z2 - 1),
        dd * (5 * z2 - 3) * z,
        c_ * x * (5 * z2 - 1),
        0.5 * b_ * z * (x2 - y2),
        a_ * x * (x2 - y2),
        e_ * x * y * (x2 - y2),
        f_ * y * z * (3 * x2 - y2),
        g_ * x * y * (7 * z2 - 1),
        h_ * y * z * (7 * z2 - 3),
        0.375 * (35 * z2 * z2 - 30 * z2 + 3),
        h_ * x * z * (7 * z2 - 3),
        0.375 * s5 * (x2 - y2) * (7 * z2 - 1),
        f_ * x * z * (x2 - y2),
        (3.0 / 16.0) * math.sqrt(35.0) * (x2 * x2 - 6 * x2 * y2 + y2 * y2),
    ]
    shW = jnp.stack(comps, axis=0)
    sh_sl = [
        [
            jnp.broadcast_to(
                shW[:, d, a * _MB:(a + 1) * _MB][:, :, None],
                (_SH, _MB, _HS),
            )
            for a in range(_S)
        ]
        for d in range(_S - 1)
    ]

    kk = jax.lax.broadcasted_iota(jnp.int32, (1, _K), 1).astype(jnp.float32)
    logxC = _col(logx)
    log1mxC = _col(log1mx)
    fcutC = _col(fcut)
    validC = _col(valid)
    rbf = jnp.exp(logb + kk * logxC + (_K - 1 - kk) * log1mxC) * fcutC
    return rbf, validC, sh_sl


def _body(posT_ref, an_ref, logb_ref, emb_ref, W1_ref, b1_ref, W2_ref,
          b2_ref, Wlin_ref, out_ref):
    rbf, validC, sh_sl = _edge_geometry(posT_ref[:], logb_ref[:])

    an = an_ref[:]
    tt = jax.lax.broadcasted_iota(jnp.int32, (1, _NUM_TYPES), 1)
    oh = (an == tt).astype(jnp.float32)
    node_attr = jnp.dot(oh, emb_ref[:], preferred_element_type=jnp.float32)

    zeros_tail = jnp.zeros((_SH - 1, _MB, _HS), jnp.float32)
    X0 = tuple(
        jnp.concatenate(
            [node_attr[None, a * _MB:(a + 1) * _MB, :], zeros_tail], axis=0
        )
        for a in range(_S)
    )

    inv = 1.0 / float(_S - 1)

    def layer(i, X):
        h = jnp.maximum(
            jnp.dot(rbf, W1_ref[i], preferred_element_type=jnp.float32)
            + b1_ref[i],
            0.0,
        )
        w = (
            jnp.dot(h, W2_ref[i], preferred_element_type=jnp.float32)
            + b2_ref[i]
        ) * validC
        Xn = []
        for a in range(_S):
            acc = None
            for d in range(1, _S):
                b = (a + d) % _S
                row = (d - 1) * _NB + a * _MB
                w1g = w[row:row + _MB, :_HS]
                w2g = w[row:row + _MB, _HS:]
                t = w1g[None] * X[b] + sh_sl[d - 1][a] * w2g[None]
                acc = t if acc is None else acc + t
            new = jnp.dot(
                (acc * inv).reshape(_SH * _MB, _HS),
                Wlin_ref[i],
                preferred_element_type=jnp.float32,
            ).reshape(_SH, _MB, _HS)
            Xn.append(new)
        alpha = jnp.where(i > 0, 1.0, 0.0).astype(jnp.float32)
        out = []
        for a in range(_S):
            xn = alpha * X[a] + Xn[a]
            sc = xn[0]
            gate = jax.nn.sigmoid(sc)
            head = jax.nn.softplus(sc) - math.log(2.0)
            out.append(
                jnp.concatenate([head[None], xn[1:] * gate[None]], axis=0)
            )
        return tuple(out)

    Xf = jax.lax.fori_loop(0, _L, layer, X0)
    for a in range(_S):
        out_ref[:, a * _MB:(a + 1) * _MB, :] = Xf[a]


def kernel(pos, atomic_numbers, batch, molecule_size, emb, W1, b1, W2, b2,
           Wlin):
    del batch, molecule_size
    posT = jnp.transpose(
        pos.reshape(_G, _MB, _S, 3), (3, 0, 2, 1)
    ).reshape(3, _N)
    anP = jnp.transpose(
        atomic_numbers.reshape(_G, _MB, _S), (0, 2, 1)
    ).reshape(_N, 1).astype(jnp.int32)
    b1r = b1.reshape(_L, 1, _HS)
    b2r = b2.reshape(_L, 1, 2 * _HS)
    logb = jnp.asarray(_LOGBINOM).reshape(1, _K)
    res = pl.pallas_call(
        _body,
        grid=(_G,),
        in_specs=[
            pl.BlockSpec((3, _NB), lambda g: (0, g)),
            pl.BlockSpec((_NB, 1), lambda g: (g, 0)),
            pl.BlockSpec((1, _K), lambda g: (0, 0)),
            pl.BlockSpec((_NUM_TYPES, _HS), lambda g: (0, 0)),
            pl.BlockSpec((_L, _K, _HS), lambda g: (0, 0, 0)),
            pl.BlockSpec((_L, 1, _HS), lambda g: (0, 0, 0)),
            pl.BlockSpec((_L, _HS, 2 * _HS), lambda g: (0, 0, 0)),
            pl.BlockSpec((_L, 1, 2 * _HS), lambda g: (0, 0, 0)),
            pl.BlockSpec((_L, _HS, _HS), lambda g: (0, 0, 0)),
        ],
        out_specs=pl.BlockSpec((_SH, _NB, _HS), lambda g: (0, g, 0)),
        out_shape=jax.ShapeDtypeStruct((_SH, _N, _HS), jnp.float32),
        compiler_params=pltpu.CompilerParams(
            dimension_semantics=("parallel",)
        ),
    )(posT, anP, logb, emb, W1, b1r, W2, b2r, Wlin)
    return jnp.transpose(
        res.reshape(_SH, _G, _S, _MB, _HS), (1, 3, 2, 4, 0)
    ).reshape(_N, _HS, _SH)


# speedup vs baseline: 41.0509x; 1.1060x over previous
import math

import jax
import jax.numpy as jnp
import numpy as np
from jax.experimental import pallas as pl
from jax.experimental.pallas import tpu as pltpu

_M = 128
_S = 8
_N = _M * _S
_HS = 128
_K = 32
_L = 5
_SH = 25
_CUT = 15.0
_ALPHA = 0.5
_NUM_TYPES = 20

_MB = 32
_NB = _MB * _S
_G = _N // _NB
_E = (_S - 1) * _NB

_LOGBINOM = np.log(
    np.array([math.comb(_K - 1, k) for k in range(_K)], dtype=np.float64)
).astype(np.float32)


def _col(rows):
    parts = [jnp.transpose(rows[d:d + 1, :]) for d in range(_S - 1)]
    return jnp.concatenate(parts, axis=0)


def _flat(rows):
    parts = [rows[d:d + 1, :] for d in range(_S - 1)]
    return jnp.concatenate(parts, axis=1)


def _edge_geometry(posT, logb):
    ev_rows = []
    for d in range(1, _S):
        src = jnp.roll(posT, -_MB * d, axis=1)
        ev_rows.append(src - posT)
    ex = jnp.concatenate([e[0:1] for e in ev_rows], axis=0)
    ey = jnp.concatenate([e[1:2] for e in ev_rows], axis=0)
    ez = jnp.concatenate([e[2:3] for e in ev_rows], axis=0)
    r2 = ex * ex + ey * ey + ez * ez
    r = jnp.sqrt(r2)
    valid = (r < _CUT).astype(jnp.float32)
    xb = jnp.exp(-_ALPHA * r)
    logx = jnp.log(xb + 1e-10)
    log1mx = jnp.log(1.0 - xb + 1e-10)
    fcut = jnp.where(
        r < _CUT, jnp.exp(-r2 / ((_CUT - r) * (_CUT + r) + 1e-9)), 0.0
    )
    rinv = 1.0 / (r + 1e-9)
    x = ey * rinv
    y = ez * rinv
    z = ex * rinv
    x2 = x * x
    y2 = y * y
    z2 = z * z
    s3 = math.sqrt(3.0)
    s5 = math.sqrt(5.0)
    s15 = math.sqrt(15.0)
    a_ = math.sqrt(35.0 / 8.0)
    b_ = math.sqrt(105.0)
    c_ = math.sqrt(21.0 / 8.0)
    dd = math.sqrt(7.0) / 2.0
    e_ = 0.75 * math.sqrt(35.0)
    f_ = 0.75 * math.sqrt(17.5)
    g_ = 0.75 * s5
    h_ = 0.75 * math.sqrt(2.5)
    comps = [
        jnp.ones_like(x),
        s3 * x,
        s3 * y,
        s3 * z,
        s15 * x * y,
        s15 * y * z,
        0.5 * s5 * (3 * z2 - 1),
        s15 * x * z,
        0.5 * s15 * (x2 - y2),
        a_ * y * (3 * x2 - y2),
        b_ * x * y * z,
        c_ * y * (5 * z2 - 1),
        dd * (5 * z2 - 3) * z,
        c_ * x * (5 * z2 - 1),
        0.5 * b_ * z * (x2 - y2),
        a_ * x * (x2 - y2),
        e_ * x * y * (x2 - y2),
        f_ * y * z * (3 * x2 - y2),
        g_ * x * y * (7 * z2 - 1),
        h_ * y * z * (7 * z2 - 3),
        0.375 * (35 * z2 * z2 - 30 * z2 + 3),
        h_ * x * z * (7 * z2 - 3),
        0.375 * s5 * (x2 - y2) * (7 * z2 - 1),
        f_ * x * z * (x2 - y2),
        (3.0 / 16.0) * math.sqrt(35.0) * (x2 * x2 - 6 * x2 * y2 + y2 * y2),
    ]
    shW = jnp.stack(comps, axis=0)
    sh_d = [shW[:, d, :] for d in range(_S - 1)]

    kk = jax.lax.broadcasted_iota(jnp.int32, (1, _K), 1).astype(jnp.float32)
    kkT = jnp.transpose(kk)
    logbT = jnp.transpose(logb)
    logxC = _col(logx)
    log1mxC = _col(log1mx)
    fcutC = _col(fcut)
    validC = _col(valid)
    rbf = jnp.exp(logb + kk * logxC + (_K - 1 - kk) * log1mxC) * fcutC
    logxF = _flat(logx)
    log1mxF = _flat(log1mx)
    fcutF = _flat(fcut)
    validF = _flat(valid)
    rbfT = jnp.exp(logbT + kkT * logxF + (_K - 1 - kkT) * log1mxF) * fcutF
    return rbf, validC, rbfT, validF, sh_d


def _body(posT_ref, an_ref, logb_ref, emb_ref, W1_ref, b1_ref, W2a_ref,
          b2a_ref, W1T_ref, b1T_ref, W2bT_ref, b2bT_ref, WlinS_ref, out_ref):
    rbf, validC, rbfT, validF, sh_d = _edge_geometry(
        posT_ref[:], logb_ref[:]
    )

    an = an_ref[:]
    tt = jax.lax.broadcasted_iota(jnp.int32, (1, _NUM_TYPES), 1)
    oh = (an == tt).astype(jnp.float32)
    node_attr = jnp.dot(oh, emb_ref[:], preferred_element_type=jnp.float32)

    zeros_tail = jnp.zeros((_SH - 1, _MB, _HS), jnp.float32)
    X0 = tuple(
        jnp.concatenate(
            [node_attr[None, a * _MB:(a + 1) * _MB, :], zeros_tail], axis=0
        )
        for a in range(_S)
    )

    def layer(i, X):
        h = jnp.maximum(
            jnp.dot(rbf, W1_ref[i], preferred_element_type=jnp.float32)
            + b1_ref[i],
            0.0,
        )
        w1 = (
            jnp.dot(h, W2a_ref[i], preferred_element_type=jnp.float32)
            + b2a_ref[i]
        ) * validC
        hT = jnp.maximum(
            jnp.dot(W1T_ref[i], rbfT, preferred_element_type=jnp.float32)
            + b1T_ref[i],
            0.0,
        )
        w2T = (
            jnp.dot(W2bT_ref[i], hT, preferred_element_type=jnp.float32)
            + b2bT_ref[i]
        ) * validF
        t2 = None
        for d in range(1, _S):
            sl = w2T[:, (d - 1) * _NB:d * _NB]
            term = sh_d[d - 1][:, None, :] * sl[None]
            t2 = term if t2 is None else t2 + term
        t2new = jax.lax.dot_general(
            t2, WlinS_ref[i], (((1,), (0,)), ((), ())),
            preferred_element_type=jnp.float32,
        )
        alpha = jnp.where(i > 0, 1.0, 0.0).astype(jnp.float32)
        out = []
        for a in range(_S):
            acc = None
            for d in range(1, _S):
                b = (a + d) % _S
                row = (d - 1) * _NB + a * _MB
                t = w1[row:row + _MB, :][None] * X[b]
                acc = t if acc is None else acc + t
            new = jnp.dot(
                acc.reshape(_SH * _MB, _HS),
                WlinS_ref[i],
                preferred_element_type=jnp.float32,
            ).reshape(_SH, _MB, _HS) + t2new[:, a * _MB:(a + 1) * _MB, :]
            xn = alpha * X[a] + new
            sc = xn[0]
            gate = jax.nn.sigmoid(sc)
            head = jax.nn.softplus(sc) - math.log(2.0)
            out.append(
                jnp.concatenate([head[None], xn[1:] * gate[None]], axis=0)
            )
        return tuple(out)

    Xf = jax.lax.fori_loop(0, _L, layer, X0)
    for a in range(_S):
        out_ref[:, a * _MB:(a + 1) * _MB, :] = Xf[a]


def kernel(pos, atomic_numbers, batch, molecule_size, emb, W1, b1, W2, b2,
           Wlin):
    del batch, molecule_size
    posT = jnp.transpose(
        pos.reshape(_G, _MB, _S, 3), (3, 0, 2, 1)
    ).reshape(3, _N)
    anP = jnp.transpose(
        atomic_numbers.reshape(_G, _MB, _S), (0, 2, 1)
    ).reshape(_N, 1).astype(jnp.int32)
    b1r = b1.reshape(_L, 1, _HS)
    W2a = W2[:, :, :_HS]
    b2a = b2[:, :_HS].reshape(_L, 1, _HS)
    W1T = jnp.transpose(W1, (0, 2, 1))
    b1T = b1.reshape(_L, _HS, 1)
    W2bT = jnp.transpose(W2[:, :, _HS:], (0, 2, 1))
    b2bT = b2[:, _HS:].reshape(_L, _HS, 1)
    WlinS = Wlin * (1.0 / float(_S - 1))
    logb = jnp.asarray(_LOGBINOM).reshape(1, _K)
    res = pl.pallas_call(
        _body,
        grid=(_G,),
        in_specs=[
            pl.BlockSpec((3, _NB), lambda g: (0, g)),
            pl.BlockSpec((_NB, 1), lambda g: (g, 0)),
            pl.BlockSpec((1, _K), lambda g: (0, 0)),
            pl.BlockSpec((_NUM_TYPES, _HS), lambda g: (0, 0)),
            pl.BlockSpec((_L, _K, _HS), lambda g: (0, 0, 0)),
            pl.BlockSpec((_L, 1, _HS), lambda g: (0, 0, 0)),
            pl.BlockSpec((_L, _HS, _HS), lambda g: (0, 0, 0)),
            pl.BlockSpec((_L, 1, _HS), lambda g: (0, 0, 0)),
            pl.BlockSpec((_L, _HS, _K), lambda g: (0, 0, 0)),
            pl.BlockSpec((_L, _HS, 1), lambda g: (0, 0, 0)),
            pl.BlockSpec((_L, _HS, _HS), lambda g: (0, 0, 0)),
            pl.BlockSpec((_L, _HS, 1), lambda g: (0, 0, 0)),
            pl.BlockSpec((_L, _HS, _HS), lambda g: (0, 0, 0)),
        ],
        out_specs=pl.BlockSpec((_SH, _NB, _HS), lambda g: (0, g, 0)),
        out_shape=jax.ShapeDtypeStruct((_SH, _N, _HS), jnp.float32),
        compiler_params=pltpu.CompilerParams(
            dimension_semantics=("parallel",)
        ),
    )(posT, anP, logb, emb, W1, b1r, W2a, b2a, W1T, b1T, W2bT, b2bT, WlinS)
    return jnp.transpose(
        res.reshape(_SH, _G, _S, _MB, _HS), (1, 3, 2, 4, 0)
    ).reshape(_N, _HS, _SH)
